# Initial kernel scaffold; baseline (speedup 1.0000x reference)
import jax
import jax.numpy as jnp
from jax.experimental import pallas as pl


def kernel(h_node, x, ptr, W, b):
    raise NotImplementedError("write your pallas kernel here")


# speedup vs baseline: 99.1665x; 99.1665x over previous
import functools

import jax
import jax.numpy as jnp
from jax import lax
from jax.experimental import pallas as pl
from jax.experimental.pallas import tpu as pltpu
from jax.experimental.pallas import tpu_sc as plsc

_NB = 2560
_SEG_PER_W = 320
_PTR_CHUNK = 328


def _mlp_cumsum_block(x_ref, h_ref, w_ref, b_ref, out_ref, acc_ref, *,
                      steps, nb, f, d):
    i = pl.program_id(0)

    @pl.when(i == 0)
    def _init():
        acc_ref[...] = jnp.zeros_like(acc_ref)

    acc = acc_ref[0:1, :]

    @pl.when(i < steps)
    def _body():
        w = w_ref[...]
        z = (jnp.dot(x_ref[...], w[:f], preferred_element_type=jnp.float32)
             + jnp.dot(h_ref[...], w[f:], preferred_element_type=jnp.float32)
             + b_ref[...])
        z = jnp.where(z >= 0, z, 0.2 * z)
        s = z
        k = 1
        while k < nb:
            s = s + jnp.concatenate(
                [jnp.zeros((k, d), jnp.float32), s[:-k]], axis=0)
            k *= 2
        out_ref[...] = acc + (s - z)
        acc_ref[...] = jnp.broadcast_to(acc + s[nb - 1:nb, :], acc_ref.shape)

    @pl.when(i == steps)
    def _tail():
        out_ref[...] = jnp.broadcast_to(acc, out_ref.shape)


def _mlp_cumsum(x, h_node, w, b2d, *, nb):
    n, f = x.shape
    d = h_node.shape[1]
    steps = n // nb
    return pl.pallas_call(
        functools.partial(_mlp_cumsum_block, steps=steps, nb=nb, f=f, d=d),
        grid=(steps + 1,),
        in_specs=[
            pl.BlockSpec((nb, f), lambda i: (jnp.minimum(i, steps - 1), 0)),
            pl.BlockSpec((nb, d), lambda i: (jnp.minimum(i, steps - 1), 0)),
            pl.BlockSpec((f + d, d), lambda i: (0, 0)),
            pl.BlockSpec((1, d), lambda i: (0, 0)),
        ],
        out_specs=pl.BlockSpec((nb, d), lambda i: (i, 0)),
        out_shape=jax.ShapeDtypeStruct(((steps + 1) * nb, d), jnp.float32),
        scratch_shapes=[pltpu.VMEM((8, d), jnp.float32)],
        compiler_params=pltpu.CompilerParams(
            dimension_semantics=("arbitrary",)),
    )(x, h_node, w, b2d)


def _seg_diff_body(s_hbm, ptr_hbm, out_hbm, idx_v, buf_v, out_v, sem, *,
                   nc, d):
    wid = lax.axis_index("s") * nc + lax.axis_index("c")
    base = wid * _SEG_PER_W
    pltpu.sync_copy(ptr_hbm.at[pl.ds(base, _PTR_CHUNK)], idx_v)
    copies = []
    for off, ln in ((0, 112), (112, 112), (224, 104)):
        copies.append(pltpu.async_copy(
            s_hbm.at[idx_v.at[pl.ds(off, ln)]], buf_v.at[pl.ds(off, ln)],
            sem))
    for c in copies:
        c.wait()

    def body(j, carry):
        for c0 in range(0, d, 16):
            lo = buf_v[j, pl.ds(c0, 16)]
            hi = buf_v[j + 1, pl.ds(c0, 16)]
            out_v[j, pl.ds(c0, 16)] = hi - lo
        return carry

    lax.fori_loop(0, _SEG_PER_W, body, 0)
    pltpu.sync_copy(out_v, out_hbm.at[pl.ds(base, _SEG_PER_W)])


def _seg_diff(s_ex, ptr_pad, *, d):
    info = plsc.get_sparse_core_info()
    nc, ns = info.num_cores, info.num_subcores
    nw = nc * ns
    mesh = plsc.VectorSubcoreMesh(core_axis_name="c", subcore_axis_name="s")
    kern = functools.partial(
        pl.kernel,
        mesh=mesh,
        out_type=jax.ShapeDtypeStruct((nw * _SEG_PER_W, d), jnp.float32),
        scratch_types=[
            pltpu.VMEM((_PTR_CHUNK,), jnp.int32),
            pltpu.VMEM((_PTR_CHUNK, d), jnp.float32),
            pltpu.VMEM((_SEG_PER_W, d), jnp.float32),
            pltpu.SemaphoreType.DMA,
        ],
        compiler_params=pltpu.CompilerParams(use_tc_tiling_on_sc=False),
    )(functools.partial(_seg_diff_body, nc=nc, d=d))
    return kern(s_ex, ptr_pad)


def kernel(h_node, x, ptr, W, b):
    n, d = h_node.shape
    nseg = ptr.shape[0] - 1
    s_ex = _mlp_cumsum(x, h_node, W, b.reshape(1, d), nb=_NB)
    info = plsc.get_sparse_core_info()
    nw = info.num_cores * info.num_subcores
    pad_len = nw * _SEG_PER_W + (_PTR_CHUNK - _SEG_PER_W) - (nseg + 1)
    ptr_pad = jnp.concatenate(
        [ptr, jnp.full((pad_len,), n, dtype=ptr.dtype)])
    out = _seg_diff(s_ex, ptr_pad, d=d)
    return out[:nseg]


# speedup vs baseline: 103.2358x; 1.0410x over previous
import functools

import jax
import jax.numpy as jnp
from jax import lax
from jax.experimental import pallas as pl
from jax.experimental.pallas import tpu as pltpu
from jax.experimental.pallas import tpu_sc as plsc

_NB = 5120
_SEG_PER_W = 320
_PTR_CHUNK = 328


_SUB = 512


def _mlp_cumsum_block(x_ref, h_ref, w_ref, b_ref, out_ref, acc_ref, *,
                      steps, nb, f, d):
    i = pl.program_id(0)

    @pl.when(i == 0)
    def _init():
        acc_ref[...] = jnp.zeros_like(acc_ref)

    acc = acc_ref[0:1, :]

    @pl.when(i < steps)
    def _body():
        w = w_ref[...]
        z = (jnp.dot(x_ref[...], w[:f], preferred_element_type=jnp.float32)
             + jnp.dot(h_ref[...], w[f:], preferred_element_type=jnp.float32)
             + b_ref[...])
        z = jnp.where(z >= 0, z, 0.2 * z)
        ltri = (lax.broadcasted_iota(jnp.int32, (_SUB, _SUB), 0)
                > lax.broadcasted_iota(jnp.int32, (_SUB, _SUB), 1)
                ).astype(jnp.float32)
        ones_row = jnp.ones((1, _SUB), jnp.float32)
        offs = acc
        outs = []
        for s0 in range(0, nb, _SUB):
            zs = z[s0:s0 + _SUB]
            ex = jnp.dot(ltri, zs, preferred_element_type=jnp.float32)
            outs.append(offs + ex)
            offs = offs + jnp.dot(ones_row, zs,
                                  preferred_element_type=jnp.float32)
        out_ref[...] = jnp.concatenate(outs, axis=0)
        acc_ref[...] = jnp.broadcast_to(offs, acc_ref.shape)

    @pl.when(i == steps)
    def _tail():
        out_ref[...] = jnp.broadcast_to(acc, out_ref.shape)


def _mlp_cumsum(x, h_node, w, b2d, *, nb):
    n, f = x.shape
    d = h_node.shape[1]
    steps = n // nb
    return pl.pallas_call(
        functools.partial(_mlp_cumsum_block, steps=steps, nb=nb, f=f, d=d),
        grid=(steps + 1,),
        in_specs=[
            pl.BlockSpec((nb, f), lambda i: (jnp.minimum(i, steps - 1), 0)),
            pl.BlockSpec((nb, d), lambda i: (jnp.minimum(i, steps - 1), 0)),
            pl.BlockSpec((f + d, d), lambda i: (0, 0)),
            pl.BlockSpec((1, d), lambda i: (0, 0)),
        ],
        out_specs=pl.BlockSpec((nb, d), lambda i: (i, 0)),
        out_shape=jax.ShapeDtypeStruct(((steps + 1) * nb, d), jnp.float32),
        scratch_shapes=[pltpu.VMEM((8, d), jnp.float32)],
        compiler_params=pltpu.CompilerParams(
            dimension_semantics=("arbitrary",)),
    )(x, h_node, w, b2d)


def _seg_diff_body(s_hbm, ptr_hbm, out_hbm, idx_v, buf_v, out_v, sem, *,
                   nc, d):
    wid = lax.axis_index("s") * nc + lax.axis_index("c")
    base = wid * _SEG_PER_W
    pltpu.sync_copy(ptr_hbm.at[pl.ds(base, _PTR_CHUNK)], idx_v)
    copies = []
    for off, ln in ((0, 112), (112, 112), (224, 104)):
        copies.append(pltpu.async_copy(
            s_hbm.at[idx_v.at[pl.ds(off, ln)]], buf_v.at[pl.ds(off, ln)],
            sem))
    for c in copies:
        c.wait()

    def body(j, carry):
        for c0 in range(0, d, 16):
            lo = buf_v[j, pl.ds(c0, 16)]
            hi = buf_v[j + 1, pl.ds(c0, 16)]
            out_v[j, pl.ds(c0, 16)] = hi - lo
        return carry

    lax.fori_loop(0, _SEG_PER_W, body, 0)
    pltpu.sync_copy(out_v, out_hbm.at[pl.ds(base, _SEG_PER_W)])


def _seg_diff(s_ex, ptr_pad, *, d):
    info = plsc.get_sparse_core_info()
    nc, ns = info.num_cores, info.num_subcores
    nw = nc * ns
    mesh = plsc.VectorSubcoreMesh(core_axis_name="c", subcore_axis_name="s")
    kern = functools.partial(
        pl.kernel,
        mesh=mesh,
        out_type=jax.ShapeDtypeStruct((nw * _SEG_PER_W, d), jnp.float32),
        scratch_types=[
            pltpu.VMEM((_PTR_CHUNK,), jnp.int32),
            pltpu.VMEM((_PTR_CHUNK, d), jnp.float32),
            pltpu.VMEM((_SEG_PER_W, d), jnp.float32),
            pltpu.SemaphoreType.DMA,
        ],
        compiler_params=pltpu.CompilerParams(use_tc_tiling_on_sc=False),
    )(functools.partial(_seg_diff_body, nc=nc, d=d))
    return kern(s_ex, ptr_pad)


def kernel(h_node, x, ptr, W, b):
    n, d = h_node.shape
    nseg = ptr.shape[0] - 1
    s_ex = _mlp_cumsum(x, h_node, W, b.reshape(1, d), nb=_NB)
    info = plsc.get_sparse_core_info()
    nw = info.num_cores * info.num_subcores
    pad_len = nw * _SEG_PER_W + (_PTR_CHUNK - _SEG_PER_W) - (nseg + 1)
    ptr_pad = jnp.concatenate(
        [ptr, jnp.full((pad_len,), n, dtype=ptr.dtype)])
    out = _seg_diff(s_ex, ptr_pad, d=d)
    return out[:nseg]


# speedup vs baseline: 135.0792x; 1.3085x over previous
import functools

import jax
import jax.numpy as jnp
from jax import lax
from jax.experimental import pallas as pl
from jax.experimental.pallas import tpu as pltpu
from jax.experimental.pallas import tpu_sc as plsc

_NB = 5120
_SEG_PER_W = 320
_PTR_CHUNK = 328


_SUB = 512


def _mlp_cumsum_block(x_ref, h_ref, w_ref, b_ref, out_ref, acc_ref, *,
                      steps, nb, f, d):
    i = pl.program_id(0)

    @pl.when(i == 0)
    def _init():
        acc_ref[...] = jnp.zeros_like(acc_ref)

    acc = acc_ref[0:1, :]

    @pl.when(i < steps)
    def _body():
        w = w_ref[...]
        z = (jnp.dot(x_ref[...], w[:f], preferred_element_type=jnp.float32)
             + jnp.dot(h_ref[...], w[f:], preferred_element_type=jnp.float32)
             + b_ref[...])
        z = jnp.where(z >= 0, z, 0.2 * z)
        ltri = (lax.broadcasted_iota(jnp.int32, (_SUB, _SUB), 0)
                > lax.broadcasted_iota(jnp.int32, (_SUB, _SUB), 1)
                ).astype(jnp.float32)
        ones_row = jnp.ones((1, _SUB), jnp.float32)
        offs = acc
        outs = []
        for s0 in range(0, nb, _SUB):
            zs = z[s0:s0 + _SUB]
            ex = jnp.dot(ltri, zs, preferred_element_type=jnp.float32)
            outs.append(offs + ex)
            offs = offs + jnp.dot(ones_row, zs,
                                  preferred_element_type=jnp.float32)
        out_ref[:, 0:d] = jnp.concatenate(outs, axis=0)
        acc_ref[...] = jnp.broadcast_to(offs, acc_ref.shape)

    @pl.when(i == steps)
    def _tail():
        out_ref[:, 0:d] = jnp.broadcast_to(acc, (nb, d))


def _mlp_cumsum(x, h_node, w, b2d, *, nb):
    n, f = x.shape
    d = h_node.shape[1]
    steps = n // nb
    return pl.pallas_call(
        functools.partial(_mlp_cumsum_block, steps=steps, nb=nb, f=f, d=d),
        grid=(steps + 1,),
        in_specs=[
            pl.BlockSpec((nb, f), lambda i: (jnp.minimum(i, steps - 1), 0)),
            pl.BlockSpec((nb, d), lambda i: (jnp.minimum(i, steps - 1), 0)),
            pl.BlockSpec((f + d, d), lambda i: (0, 0)),
            pl.BlockSpec((1, d), lambda i: (0, 0)),
        ],
        out_specs=pl.BlockSpec((nb, 128), lambda i: (i, 0)),
        out_shape=jax.ShapeDtypeStruct(((steps + 1) * nb, 128), jnp.float32),
        scratch_shapes=[pltpu.VMEM((8, d), jnp.float32)],
        compiler_params=pltpu.CompilerParams(
            dimension_semantics=("arbitrary",)),
    )(x, h_node, w, b2d)


def _seg_diff_body(s_hbm, ptr_hbm, out_hbm, idx_v, buf_v, out_v, sem, *,
                   nc, d):
    wid = lax.axis_index("s") * nc + lax.axis_index("c")
    base = wid * _SEG_PER_W
    pltpu.sync_copy(ptr_hbm.at[pl.ds(base, _PTR_CHUNK)], idx_v)
    copies = []
    for off, ln in ((0, 112), (112, 112), (224, 104)):
        copies.append(pltpu.async_copy(
            s_hbm.at[idx_v.at[pl.ds(off, ln)]], buf_v.at[pl.ds(off, ln)],
            sem))
    for c in copies:
        c.wait()

    def body(j, carry):
        for c0 in range(0, d, 16):
            lo = buf_v[j, pl.ds(c0, 16)]
            hi = buf_v[j + 1, pl.ds(c0, 16)]
            out_v[j, pl.ds(c0, 16)] = hi - lo
        return carry

    lax.fori_loop(0, _SEG_PER_W, body, 0)
    pltpu.sync_copy(out_v, out_hbm.at[pl.ds(base, _SEG_PER_W)])


def _seg_diff(s_ex, ptr_pad, *, d):
    info = plsc.get_sparse_core_info()
    nc, ns = info.num_cores, info.num_subcores
    nw = nc * ns
    mesh = plsc.VectorSubcoreMesh(core_axis_name="c", subcore_axis_name="s")
    kern = functools.partial(
        pl.kernel,
        mesh=mesh,
        out_type=jax.ShapeDtypeStruct((nw * _SEG_PER_W, d), jnp.float32),
        scratch_types=[
            pltpu.VMEM((_PTR_CHUNK,), jnp.int32),
            pltpu.VMEM((_PTR_CHUNK, 128), jnp.float32),
            pltpu.VMEM((_SEG_PER_W, d), jnp.float32),
            pltpu.SemaphoreType.DMA,
        ],
        compiler_params=pltpu.CompilerParams(use_tc_tiling_on_sc=False),
    )(functools.partial(_seg_diff_body, nc=nc, d=d))
    return kern(s_ex, ptr_pad)


def kernel(h_node, x, ptr, W, b):
    n, d = h_node.shape
    nseg = ptr.shape[0] - 1
    s_ex = _mlp_cumsum(x, h_node, W, b.reshape(1, d), nb=_NB)
    info = plsc.get_sparse_core_info()
    nw = info.num_cores * info.num_subcores
    pad_len = nw * _SEG_PER_W + (_PTR_CHUNK - _SEG_PER_W) - (nseg + 1)
    ptr_pad = jnp.concatenate(
        [ptr, jnp.full((pad_len,), n, dtype=ptr.dtype)])
    out = _seg_diff(s_ex, ptr_pad, d=d)
    return out[:nseg]
